# Initial kernel scaffold; baseline (speedup 1.0000x reference)
import jax
import jax.numpy as jnp
from jax.experimental import pallas as pl


def kernel(x, edge_index, batch, W_embed, b_embed, gcn_W, gcn_b, bn_gamma, bn_beta, head_W1, head_b1, head_W2, head_b2):
    raise NotImplementedError("write your pallas kernel here")


# speedup vs baseline: 10.7766x; 10.7766x over previous
import functools

import jax
import jax.numpy as jnp
from jax import lax
from jax.experimental import pallas as pl
from jax.experimental.pallas import tpu as pltpu
from jax.experimental.pallas import tpu_sc as plsc

_N = 10000
_NP = 10240
_E = 320000
_H = 128
_NGRAPH = 16
_EPS = 1e-5

_NC = 2
_NS = 16
_EPT = _E // (_NC * _NS)
_EC = 80
_NEC = _EPT // _EC
_RPT = _NP // _NS
_RC = 128
_NRC = _RPT // _RC
_DW = 16

_MESH = plsc.VectorSubcoreMesh(core_axis_name="c", subcore_axis_name="s",
                               num_cores=_NC, num_subcores=_NS)

_HIGH = lax.Precision.HIGHEST
_TC_PARAMS = pltpu.CompilerParams(vmem_limit_bytes=100 * 1024 * 1024)


@functools.partial(
    pl.kernel,
    out_type=jax.ShapeDtypeStruct((_NC, _NP, _DW), jnp.float32),
    mesh=_MESH,
    scratch_types=[
        pltpu.VMEM_SHARED((_NP, _DW), jnp.float32),
        pltpu.VMEM((_EC,), jnp.int32),
        pltpu.VMEM((_EC, _DW), jnp.float32),
        pltpu.VMEM((_RPT, _DW), jnp.float32),
    ],
)
def _deg_scatter(dst_hbm, deg_out, deg_sp, idx_v, ones_v, stage_v):
    c = lax.axis_index("c")
    s = lax.axis_index("s")
    row0 = s * _RPT

    def fill(i, carry):
        ones_v[i, :] = jnp.full((_DW,), 1.0, jnp.float32)
        return carry

    lax.fori_loop(0, _EC, fill, 0)

    def zfill(i, carry):
        stage_v[i, :] = jnp.zeros((_DW,), jnp.float32)
        return carry

    lax.fori_loop(0, _RPT, zfill, 0)
    pltpu.sync_copy(stage_v, deg_sp.at[pl.ds(row0, _RPT)])
    plsc.subcore_barrier()

    base = (c * _NS + s) * _EPT

    def edge(j, carry):
        pltpu.sync_copy(dst_hbm.at[pl.ds(base + j * _EC, _EC)], idx_v)
        pltpu.sync_copy(ones_v, deg_sp.at[idx_v], add=True)
        return carry

    lax.fori_loop(0, _NEC, edge, 0)
    plsc.subcore_barrier()
    pltpu.sync_copy(deg_sp.at[pl.ds(row0, _RPT)], stage_v)
    pltpu.sync_copy(stage_v, deg_out.at[c, pl.ds(row0, _RPT)])


@functools.partial(
    pl.kernel,
    out_type=jax.ShapeDtypeStruct((_NC, _NP, _H), jnp.float32),
    mesh=_MESH,
    scratch_types=[
        pltpu.VMEM_SHARED((_NP, _H), jnp.float32),
        pltpu.VMEM((_EC,), jnp.int32),
        pltpu.VMEM((_EC,), jnp.int32),
        pltpu.VMEM((_EC, _H), jnp.float32),
        pltpu.VMEM((_RC, _H), jnp.float32),
    ],
)
def _edge_scatter(hts_hbm, src_hbm, dst_hbm, acc_out, acc_sp, src_v, dst_v,
                  rows_v, stage_v):
    c = lax.axis_index("c")
    s = lax.axis_index("s")
    row0 = s * _RPT

    def init(i, carry):
        r = row0 + i * _RC
        pltpu.sync_copy(hts_hbm.at[pl.ds(r, _RC)], stage_v)
        pltpu.sync_copy(stage_v, acc_sp.at[pl.ds(r, _RC)])
        return carry

    lax.fori_loop(0, _NRC, init, 0)
    plsc.subcore_barrier()

    base = (c * _NS + s) * _EPT

    def edge(j, carry):
        eb = base + j * _EC
        pltpu.sync_copy(src_hbm.at[pl.ds(eb, _EC)], src_v)
        pltpu.sync_copy(dst_hbm.at[pl.ds(eb, _EC)], dst_v)
        pltpu.sync_copy(hts_hbm.at[src_v], rows_v)
        pltpu.sync_copy(rows_v, acc_sp.at[dst_v], add=True)
        return carry

    lax.fori_loop(0, _NEC, edge, 0)
    plsc.subcore_barrier()

    def dump(i, carry):
        r = row0 + i * _RC
        pltpu.sync_copy(acc_sp.at[pl.ds(r, _RC)], stage_v)
        pltpu.sync_copy(stage_v, acc_out.at[c, pl.ds(r, _RC)])
        return carry

    lax.fori_loop(0, _NRC, dump, 0)


def _pad_rows(a):
    return jnp.concatenate(
        [a, jnp.zeros((_NP - _N, a.shape[1]), jnp.float32)], axis=0)


def _embed_body(x_ref, wet_ref, be_ref, w0t_ref, deg_ref, h_ref, hts_ref,
                dinv_ref):
    deg = deg_ref[0, 0:_N, 0:1] + deg_ref[1, 0:_N, 0:1] + 1.0
    dinv = lax.rsqrt(deg)
    h = jnp.maximum(
        jnp.dot(x_ref[...], wet_ref[...], precision=_HIGH,
                preferred_element_type=jnp.float32) + be_ref[...], 0.0)
    h_ref[...] = h
    hts = jnp.dot(h, w0t_ref[...], precision=_HIGH,
                  preferred_element_type=jnp.float32) * dinv
    hts_ref[...] = _pad_rows(hts)
    dinv_ref[...] = dinv


_embed_tc = pl.pallas_call(
    _embed_body,
    out_shape=[
        jax.ShapeDtypeStruct((_N, _H), jnp.float32),
        jax.ShapeDtypeStruct((_NP, _H), jnp.float32),
        jax.ShapeDtypeStruct((_N, 1), jnp.float32),
    ],
    compiler_params=_TC_PARAMS,
)


def _bn_relu(acc_ref, hts_ref, h_ref, dinv_ref, b_ref, g_ref, bt_ref):
    aggp = (acc_ref[0, 0:_N, :] + acc_ref[1, 0:_N, :] - hts_ref[0:_N, :])
    agg = aggp * dinv_ref[...] + b_ref[...]
    mean = jnp.mean(agg, axis=0, keepdims=True)
    var = jnp.mean((agg - mean) ** 2, axis=0, keepdims=True)
    agg = (agg - mean) * lax.rsqrt(var + _EPS) * g_ref[...] + bt_ref[...]
    return jnp.maximum(agg, 0.0) + h_ref[...]


def _layer_body(acc_ref, hts_ref, h_ref, dinv_ref, b_ref, g_ref, bt_ref,
                wnt_ref, hout_ref, htsout_ref):
    h = _bn_relu(acc_ref, hts_ref, h_ref, dinv_ref, b_ref, g_ref, bt_ref)
    hout_ref[...] = h
    hts = jnp.dot(h, wnt_ref[...], precision=_HIGH,
                  preferred_element_type=jnp.float32) * dinv_ref[...]
    htsout_ref[...] = _pad_rows(hts)


_layer_tc = pl.pallas_call(
    _layer_body,
    out_shape=[
        jax.ShapeDtypeStruct((_N, _H), jnp.float32),
        jax.ShapeDtypeStruct((_NP, _H), jnp.float32),
    ],
    compiler_params=_TC_PARAMS,
)


def _final_body(acc_ref, hts_ref, h_ref, dinv_ref, b_ref, g_ref, bt_ref,
                batch_ref, w1t_ref, b1_ref, w2t_ref, b2_ref, out_ref):
    h = _bn_relu(acc_ref, hts_ref, h_ref, dinv_ref, b_ref, g_ref, bt_ref)
    oh = (batch_ref[...] == lax.broadcasted_iota(jnp.int32, (1, _NGRAPH), 1)
          ).astype(jnp.float32)
    sums = lax.dot_general(oh, h, (((0,), (0,)), ((), ())), precision=_HIGH,
                           preferred_element_type=jnp.float32)
    counts = lax.dot_general(oh, jnp.ones((_N, 1), jnp.float32),
                             (((0,), (0,)), ((), ())), precision=_HIGH,
                             preferred_element_type=jnp.float32)
    pooled = sums / jnp.maximum(counts, 1.0)
    z = jnp.maximum(
        jnp.dot(pooled, w1t_ref[...], precision=_HIGH,
                preferred_element_type=jnp.float32) + b1_ref[...], 0.0)
    out_ref[...] = jnp.dot(z, w2t_ref[...], precision=_HIGH,
                           preferred_element_type=jnp.float32) + b2_ref[...]


_final_tc = pl.pallas_call(
    _final_body,
    out_shape=jax.ShapeDtypeStruct((_NGRAPH, _H), jnp.float32),
    compiler_params=_TC_PARAMS,
)


def kernel(x, edge_index, batch, W_embed, b_embed, gcn_W, gcn_b, bn_gamma,
           bn_beta, head_W1, head_b1, head_W2, head_b2):
    src = edge_index[0]
    dst = edge_index[1]
    deg_parts = _deg_scatter(dst)
    h, hts, dinv = _embed_tc(x, W_embed.T, b_embed.reshape(1, _H), gcn_W[0].T,
                             deg_parts)
    out = None
    for l in range(3):
        acc = _edge_scatter(hts, src, dst)
        b_l = gcn_b[l].reshape(1, _H)
        g_l = bn_gamma[l].reshape(1, _H)
        bt_l = bn_beta[l].reshape(1, _H)
        if l < 2:
            h, hts = _layer_tc(acc, hts, h, dinv, b_l, g_l, bt_l,
                               gcn_W[l + 1].T)
        else:
            out = _final_tc(acc, hts, h, dinv, b_l, g_l, bt_l,
                            batch.reshape(_N, 1), head_W1.T,
                            head_b1.reshape(1, _H), head_W2.T,
                            head_b2.reshape(1, _H))
    return out
